# Initial kernel scaffold; baseline (speedup 1.0000x reference)
import jax
import jax.numpy as jnp
from jax.experimental import pallas as pl


def kernel(x, edge_index, edge_weight, W_in, b_in, W0, b0, W1, b1, W_out, b_out):
    raise NotImplementedError("write your pallas kernel here")


# speedup vs baseline: 9.4551x; 9.4551x over previous
import functools

import jax
import jax.numpy as jnp
from jax import lax
from jax.experimental import pallas as pl
from jax.experimental.pallas import tpu as pltpu
from jax.experimental.pallas import tpu_sc as plsc

NC = 2
NS = 16
NW = NC * NS
CHUNK = 128
LANES = 16


def _sc_degree(dst, ew, np_):
    ep = dst.shape[0]
    cpw = ep // (NW * CHUNK)
    per_sub = np_ // NS
    mesh = plsc.VectorSubcoreMesh(
        core_axis_name="c", subcore_axis_name="s", num_cores=NC, num_subcores=NS
    )

    @functools.partial(
        pl.kernel,
        out_type=jax.ShapeDtypeStruct((NC * np_,), jnp.float32),
        mesh=mesh,
        scratch_types=[
            pltpu.VMEM((CHUNK,), jnp.int32),
            pltpu.VMEM((CHUNK,), jnp.float32),
            pltpu.VMEM((per_sub,), jnp.float32),
            pltpu.VMEM_SHARED((np_,), jnp.float32),
        ],
    )
    def deg_kernel(dst_hbm, ew_hbm, out_hbm, didx, ewb, zbuf, deg_sh):
        c = lax.axis_index("c")
        s = lax.axis_index("s")
        wid = c * NS + s

        @pl.loop(0, per_sub // LANES)
        def _(j):
            zbuf[pl.ds(j * LANES, LANES)] = jnp.zeros((LANES,), jnp.float32)

        sl = pl.ds(s * per_sub, per_sub)
        pltpu.sync_copy(zbuf, deg_sh.at[sl])
        plsc.subcore_barrier()

        @pl.loop(0, cpw)
        def _(i):
            base = (wid * cpw + i) * CHUNK
            pltpu.sync_copy(dst_hbm.at[pl.ds(base, CHUNK)], didx)
            pltpu.sync_copy(ew_hbm.at[pl.ds(base, CHUNK)], ewb)
            pltpu.sync_copy(ewb, deg_sh.at[didx], add=True)

        plsc.subcore_barrier()
        pltpu.sync_copy(deg_sh.at[sl], zbuf)
        pltpu.sync_copy(zbuf, out_hbm.at[pl.ds(c * np_ + s * per_sub, per_sub)])

    return deg_kernel(dst, ew).reshape(NC, np_)


def _sc_gather_scale_scatter(g_tab, src, dst, ew_rep, np_, d):
    ep = src.shape[0]
    cpw = ep // (NW * CHUNK)
    per_sub = np_ // NS
    mesh = plsc.VectorSubcoreMesh(
        core_axis_name="c", subcore_axis_name="s", num_cores=NC, num_subcores=NS
    )

    @functools.partial(
        pl.kernel,
        out_type=jax.ShapeDtypeStruct((NC, np_, d), jnp.float32),
        mesh=mesh,
        scratch_types=[
            pltpu.VMEM((CHUNK,), jnp.int32),
            pltpu.VMEM((CHUNK,), jnp.int32),
            pltpu.VMEM((CHUNK * LANES,), jnp.float32),
            pltpu.VMEM((CHUNK, d), jnp.float32),
            pltpu.VMEM_SHARED((np_, d), jnp.float32),
            pltpu.SemaphoreType.DMA,
        ],
    )
    def layer_kernel(g_hbm, src_hbm, dst_hbm, ew_hbm, out_hbm,
                     sidx, didx, wr, rows, acc_sh, sem):
        c = lax.axis_index("c")
        s = lax.axis_index("s")
        wid = c * NS + s

        @pl.loop(0, CHUNK)
        def _(j):
            for k in range(d // LANES):
                rows[j, pl.ds(k * LANES, LANES)] = jnp.zeros((LANES,), jnp.float32)

        for k in range(per_sub // CHUNK):
            pltpu.sync_copy(rows, acc_sh.at[pl.ds(s * per_sub + k * CHUNK, CHUNK)])
        plsc.subcore_barrier()

        @pl.loop(0, cpw)
        def _(i):
            base = (wid * cpw + i) * CHUNK
            pltpu.sync_copy(src_hbm.at[pl.ds(base, CHUNK)], sidx)
            pltpu.sync_copy(dst_hbm.at[pl.ds(base, CHUNK)], didx)
            pltpu.sync_copy(ew_hbm.at[pl.ds(base * LANES, CHUNK * LANES)], wr)
            pltpu.async_copy(g_hbm.at[sidx], rows, sem).wait()

            @pl.loop(0, CHUNK)
            def _(j):
                wv = wr[pl.ds(j * LANES, LANES)]
                for k in range(d // LANES):
                    slk = pl.ds(k * LANES, LANES)
                    rows[j, slk] = rows[j, slk] * wv

            pltpu.sync_copy(rows, acc_sh.at[didx], add=True)

        plsc.subcore_barrier()
        out_core = out_hbm.at[c]
        for k in range(per_sub // CHUNK):
            sl = pl.ds(s * per_sub + k * CHUNK, CHUNK)
            pltpu.sync_copy(acc_sh.at[sl], out_core.at[sl])

    return layer_kernel(g_tab, src, dst, ew_rep)


def _dinv_from_partials(deg2):
    deg = 1.0 + deg2[:, 0] + deg2[:, 1]
    return jnp.where(deg > 0, lax.rsqrt(deg), 0.0)


def _tc_stage_a(x, degt, w_in, b_in, w0, rb):
    n, din = x.shape
    h = w0.shape[1]

    def body(deg_ref, x_ref, wi_ref, bi_ref, w0_ref, t_ref, g_ref):
        dinv = _dinv_from_partials(deg_ref[...])
        h1 = jnp.maximum(
            jnp.dot(x_ref[...], wi_ref[...], preferred_element_type=jnp.float32,
                    precision=lax.Precision.HIGHEST) + bi_ref[...], 0.0)
        t = jnp.dot(h1, w0_ref[...], preferred_element_type=jnp.float32,
                    precision=lax.Precision.HIGHEST)
        t_ref[...] = t
        g_ref[...] = dinv[:, None] * t

    return pl.pallas_call(
        body,
        grid=(n // rb,),
        in_specs=[
            pl.BlockSpec((rb, NC), lambda r: (r, 0)),
            pl.BlockSpec((rb, din), lambda r: (r, 0)),
            pl.BlockSpec((din, h), lambda r: (0, 0)),
            pl.BlockSpec((1, h), lambda r: (0, 0)),
            pl.BlockSpec((h, h), lambda r: (0, 0)),
        ],
        out_specs=[
            pl.BlockSpec((rb, h), lambda r: (r, 0)),
            pl.BlockSpec((rb, h), lambda r: (r, 0)),
        ],
        out_shape=[
            jax.ShapeDtypeStruct((n, h), jnp.float32),
            jax.ShapeDtypeStruct((n, h), jnp.float32),
        ],
    )(degt, x, w_in, b_in, w0)


def _tc_stage_b(accp, degt, t_prev, b_prev, w_next, rb):
    n, h = t_prev.shape
    hn = w_next.shape[1]

    def body(acc_ref, deg_ref, t_ref, b_ref, w_ref, tn_ref, gn_ref):
        dinv = _dinv_from_partials(deg_ref[...])
        acc = acc_ref[0] + acc_ref[1]
        hh = jnp.maximum(
            dinv[:, None] * acc + (dinv * dinv)[:, None] * t_ref[...] + b_ref[...],
            0.0)
        t = jnp.dot(hh, w_ref[...], preferred_element_type=jnp.float32,
                    precision=lax.Precision.HIGHEST)
        tn_ref[...] = t
        gn_ref[...] = dinv[:, None] * t

    return pl.pallas_call(
        body,
        grid=(n // rb,),
        in_specs=[
            pl.BlockSpec((NC, rb, h), lambda r: (0, r, 0)),
            pl.BlockSpec((rb, NC), lambda r: (r, 0)),
            pl.BlockSpec((rb, h), lambda r: (r, 0)),
            pl.BlockSpec((1, h), lambda r: (0, 0)),
            pl.BlockSpec((h, hn), lambda r: (0, 0)),
        ],
        out_specs=[
            pl.BlockSpec((rb, hn), lambda r: (r, 0)),
            pl.BlockSpec((rb, hn), lambda r: (r, 0)),
        ],
        out_shape=[
            jax.ShapeDtypeStruct((n, hn), jnp.float32),
            jax.ShapeDtypeStruct((n, hn), jnp.float32),
        ],
    )(accp, degt, t_prev, b_prev, w_next)


def _tc_stage_c(accp, degt, t_prev, b_prev, w_out, b_out, rb):
    n, h = t_prev.shape
    c = w_out.shape[1]

    def body(acc_ref, deg_ref, t_ref, b_ref, w_ref, bo_ref, o_ref):
        dinv = _dinv_from_partials(deg_ref[...])
        acc = acc_ref[0] + acc_ref[1]
        hh = jnp.maximum(
            dinv[:, None] * acc + (dinv * dinv)[:, None] * t_ref[...] + b_ref[...],
            0.0)
        o_ref[...] = jnp.dot(hh, w_ref[...], preferred_element_type=jnp.float32,
                             precision=lax.Precision.HIGHEST) + bo_ref[...]

    return pl.pallas_call(
        body,
        grid=(n // rb,),
        in_specs=[
            pl.BlockSpec((NC, rb, h), lambda r: (0, r, 0)),
            pl.BlockSpec((rb, NC), lambda r: (r, 0)),
            pl.BlockSpec((rb, h), lambda r: (r, 0)),
            pl.BlockSpec((1, h), lambda r: (0, 0)),
            pl.BlockSpec((h, c), lambda r: (0, 0)),
            pl.BlockSpec((1, c), lambda r: (0, 0)),
        ],
        out_specs=pl.BlockSpec((rb, c), lambda r: (r, 0)),
        out_shape=jax.ShapeDtypeStruct((n, c), jnp.float32),
    )(accp, degt, t_prev, b_prev, w_out, b_out)


def kernel(x, edge_index, edge_weight, W_in, b_in, W0, b0, W1, b1, W_out, b_out):
    n, _ = x.shape
    e = edge_weight.shape[0]
    rb = 1000 if n % 1000 == 0 else n // 8

    step = NW * CHUNK
    ep = ((e + step - 1) // step) * step
    pad = ep - e
    if pad:
        fill = (jnp.arange(pad, dtype=jnp.int32) * 97) % n
        src = jnp.concatenate([edge_index[0], fill])
        dst = jnp.concatenate([edge_index[1], fill])
        ew = jnp.concatenate([edge_weight, jnp.zeros((pad,), edge_weight.dtype)])
    else:
        src, dst, ew = edge_index[0], edge_index[1], edge_weight

    ew_rep = jnp.repeat(ew, LANES)

    b_in2 = b_in.reshape(1, -1)
    b02 = b0.reshape(1, -1)
    b12 = b1.reshape(1, -1)
    b_out2 = b_out.reshape(1, -1)

    np_ = ((n + NS * LANES - 1) // (NS * LANES)) * (NS * LANES)

    degp = _sc_degree(dst, ew, np_)
    degt = degp[:, :n].T

    t1, g1 = _tc_stage_a(x, degt, W_in, b_in2, W0, rb)
    acc1 = _sc_gather_scale_scatter(g1, src, dst, ew_rep, np_, g1.shape[1])
    t2, g2 = _tc_stage_b(acc1[:, :n], degt, t1, b02, W1, rb)
    acc2 = _sc_gather_scale_scatter(g2, src, dst, ew_rep, np_, g2.shape[1])
    return _tc_stage_c(acc2[:, :n], degt, t2, b12, W_out, b_out2, rb)


# speedup vs baseline: 12.4005x; 1.3115x over previous
import functools

import jax
import jax.numpy as jnp
from jax import lax
from jax.experimental import pallas as pl
from jax.experimental.pallas import tpu as pltpu
from jax.experimental.pallas import tpu_sc as plsc

NC = 2
NS = 16
NW = NC * NS
CHUNK = 128
LANES = 16


def _sc_degree(dst, ew, np_):
    ep = dst.shape[0]
    cpw = ep // (NW * CHUNK)
    per_sub = np_ // NS
    mesh = plsc.VectorSubcoreMesh(
        core_axis_name="c", subcore_axis_name="s", num_cores=NC, num_subcores=NS
    )

    @functools.partial(
        pl.kernel,
        out_type=jax.ShapeDtypeStruct((NC * np_,), jnp.float32),
        mesh=mesh,
        scratch_types=[
            pltpu.VMEM((CHUNK,), jnp.int32),
            pltpu.VMEM((CHUNK,), jnp.float32),
            pltpu.VMEM((per_sub,), jnp.float32),
            pltpu.VMEM_SHARED((np_,), jnp.float32),
        ],
    )
    def deg_kernel(dst_hbm, ew_hbm, out_hbm, didx, ewb, zbuf, deg_sh):
        c = lax.axis_index("c")
        s = lax.axis_index("s")
        wid = c * NS + s

        @pl.loop(0, per_sub // LANES)
        def _(j):
            zbuf[pl.ds(j * LANES, LANES)] = jnp.zeros((LANES,), jnp.float32)

        sl = pl.ds(s * per_sub, per_sub)
        pltpu.sync_copy(zbuf, deg_sh.at[sl])
        plsc.subcore_barrier()

        @pl.loop(0, cpw)
        def _(i):
            base = (wid * cpw + i) * CHUNK
            pltpu.sync_copy(dst_hbm.at[pl.ds(base, CHUNK)], didx)
            pltpu.sync_copy(ew_hbm.at[pl.ds(base, CHUNK)], ewb)
            pltpu.sync_copy(ewb, deg_sh.at[didx], add=True)

        plsc.subcore_barrier()
        pltpu.sync_copy(deg_sh.at[sl], zbuf)
        pltpu.sync_copy(zbuf, out_hbm.at[pl.ds(c * np_ + s * per_sub, per_sub)])

    return deg_kernel(dst, ew).reshape(NC, np_)


def _sc_gather_scale_scatter(g_tab, pidx, ew_rep, np_, d):
    cpw = pidx.shape[0] // NW
    per_sub = np_ // NS
    mesh = plsc.VectorSubcoreMesh(
        core_axis_name="c", subcore_axis_name="s", num_cores=NC, num_subcores=NS
    )

    @functools.partial(
        pl.kernel,
        out_type=jax.ShapeDtypeStruct((NC, np_, d), jnp.float32),
        mesh=mesh,
        scratch_types=[
            pltpu.VMEM((2, CHUNK), jnp.int32),
            pltpu.VMEM((2, CHUNK), jnp.int32),
            pltpu.VMEM((CHUNK * LANES,), jnp.float32),
            pltpu.VMEM((CHUNK * LANES,), jnp.float32),
            pltpu.VMEM((CHUNK, d), jnp.float32),
            pltpu.VMEM((CHUNK, d), jnp.float32),
            pltpu.VMEM_SHARED((np_, d), jnp.float32),
            pltpu.SemaphoreType.DMA,
            pltpu.SemaphoreType.DMA,
            pltpu.SemaphoreType.DMA,
        ],
    )
    def layer_kernel(g_hbm, pidx_hbm, ew_hbm, out_hbm,
                     idx0, idx1, wr0, wr1, rows0, rows1, acc_sh,
                     gsem0, gsem1, isem):
        c = lax.axis_index("c")
        s = lax.axis_index("s")
        wid = c * NS + s
        idxb = (idx0, idx1)
        wrb = (wr0, wr1)
        rowsb = (rows0, rows1)
        gsemb = (gsem0, gsem1)

        @pl.loop(0, CHUNK)
        def _(j):
            for k in range(d // LANES):
                rows0[j, pl.ds(k * LANES, LANES)] = jnp.zeros((LANES,), jnp.float32)

        for k in range(per_sub // CHUNK):
            pltpu.sync_copy(rows0, acc_sh.at[pl.ds(s * per_sub + k * CHUNK, CHUNK)])
        plsc.subcore_barrier()

        def idx_start(i, b):
            ch = wid * cpw + i
            pltpu.async_copy(pidx_hbm.at[ch], idxb[b], isem)
            pltpu.async_copy(
                ew_hbm.at[pl.ds(ch * CHUNK * LANES, CHUNK * LANES)], wrb[b], isem)

        def idx_wait(i, b):
            ch = wid * cpw + i
            pltpu.make_async_copy(pidx_hbm.at[ch], idxb[b], isem).wait()
            pltpu.make_async_copy(
                ew_hbm.at[pl.ds(ch * CHUNK * LANES, CHUNK * LANES)], wrb[b],
                isem).wait()

        def gather_start(b):
            pltpu.async_copy(g_hbm.at[idxb[b].at[0]], rowsb[b], gsemb[b])

        def gather_wait(b):
            pltpu.make_async_copy(g_hbm.at[idxb[b].at[0]], rowsb[b],
                                  gsemb[b]).wait()

        ch0 = wid * cpw
        pltpu.sync_copy(pidx_hbm.at[ch0], idx0)
        pltpu.sync_copy(ew_hbm.at[pl.ds(ch0 * CHUNK * LANES, CHUNK * LANES)], wr0)
        gather_start(0)
        idx_start(1, 1)

        @pl.loop(0, cpw // 2)
        def _(p):
            for b in (0, 1):
                i = p * 2 + b
                nb = 1 - b
                gather_wait(b)

                @pl.when(i < cpw - 1)
                def _():
                    idx_wait(i + 1, nb)
                    gather_start(nb)

                rows = rowsb[b]
                wr = wrb[b]

                @pl.loop(0, CHUNK)
                def _(j):
                    wv = wr[pl.ds(j * LANES, LANES)]
                    for k in range(d // LANES):
                        slk = pl.ds(k * LANES, LANES)
                        rows[j, slk] = rows[j, slk] * wv

                pltpu.sync_copy(rows, acc_sh.at[idxb[b].at[1]], add=True)

                @pl.when(i < cpw - 2)
                def _():
                    idx_start(i + 2, b)

        plsc.subcore_barrier()
        out_core = out_hbm.at[c]
        for k in range(per_sub // CHUNK):
            sl = pl.ds(s * per_sub + k * CHUNK, CHUNK)
            pltpu.sync_copy(acc_sh.at[sl], out_core.at[sl])

    return layer_kernel(g_tab, pidx, ew_rep)


def _dinv_from_partials(deg2):
    deg = 1.0 + deg2[:, 0] + deg2[:, 1]
    return jnp.where(deg > 0, lax.rsqrt(deg), 0.0)


def _tc_stage_a(x, degt, w_in, b_in, w0, rb):
    n, din = x.shape
    h = w0.shape[1]

    def body(deg_ref, x_ref, wi_ref, bi_ref, w0_ref, t_ref, g_ref):
        dinv = _dinv_from_partials(deg_ref[...])
        h1 = jnp.maximum(
            jnp.dot(x_ref[...], wi_ref[...], preferred_element_type=jnp.float32,
                    precision=lax.Precision.HIGHEST) + bi_ref[...], 0.0)
        t = jnp.dot(h1, w0_ref[...], preferred_element_type=jnp.float32,
                    precision=lax.Precision.HIGHEST)
        t_ref[...] = t
        g_ref[...] = dinv[:, None] * t

    return pl.pallas_call(
        body,
        grid=(n // rb,),
        in_specs=[
            pl.BlockSpec((rb, NC), lambda r: (r, 0)),
            pl.BlockSpec((rb, din), lambda r: (r, 0)),
            pl.BlockSpec((din, h), lambda r: (0, 0)),
            pl.BlockSpec((1, h), lambda r: (0, 0)),
            pl.BlockSpec((h, h), lambda r: (0, 0)),
        ],
        out_specs=[
            pl.BlockSpec((rb, h), lambda r: (r, 0)),
            pl.BlockSpec((rb, h), lambda r: (r, 0)),
        ],
        out_shape=[
            jax.ShapeDtypeStruct((n, h), jnp.float32),
            jax.ShapeDtypeStruct((n, h), jnp.float32),
        ],
    )(degt, x, w_in, b_in, w0)


def _tc_stage_b(accp, degt, t_prev, b_prev, w_next, rb):
    n, h = t_prev.shape
    hn = w_next.shape[1]

    def body(acc_ref, deg_ref, t_ref, b_ref, w_ref, tn_ref, gn_ref):
        dinv = _dinv_from_partials(deg_ref[...])
        acc = acc_ref[0] + acc_ref[1]
        hh = jnp.maximum(
            dinv[:, None] * acc + (dinv * dinv)[:, None] * t_ref[...] + b_ref[...],
            0.0)
        t = jnp.dot(hh, w_ref[...], preferred_element_type=jnp.float32,
                    precision=lax.Precision.HIGHEST)
        tn_ref[...] = t
        gn_ref[...] = dinv[:, None] * t

    return pl.pallas_call(
        body,
        grid=(n // rb,),
        in_specs=[
            pl.BlockSpec((NC, rb, h), lambda r: (0, r, 0)),
            pl.BlockSpec((rb, NC), lambda r: (r, 0)),
            pl.BlockSpec((rb, h), lambda r: (r, 0)),
            pl.BlockSpec((1, h), lambda r: (0, 0)),
            pl.BlockSpec((h, hn), lambda r: (0, 0)),
        ],
        out_specs=[
            pl.BlockSpec((rb, hn), lambda r: (r, 0)),
            pl.BlockSpec((rb, hn), lambda r: (r, 0)),
        ],
        out_shape=[
            jax.ShapeDtypeStruct((n, hn), jnp.float32),
            jax.ShapeDtypeStruct((n, hn), jnp.float32),
        ],
    )(accp, degt, t_prev, b_prev, w_next)


def _tc_stage_c(accp, degt, t_prev, b_prev, w_out, b_out, rb):
    n, h = t_prev.shape
    c = w_out.shape[1]

    def body(acc_ref, deg_ref, t_ref, b_ref, w_ref, bo_ref, o_ref):
        dinv = _dinv_from_partials(deg_ref[...])
        acc = acc_ref[0] + acc_ref[1]
        hh = jnp.maximum(
            dinv[:, None] * acc + (dinv * dinv)[:, None] * t_ref[...] + b_ref[...],
            0.0)
        o_ref[...] = jnp.dot(hh, w_ref[...], preferred_element_type=jnp.float32,
                             precision=lax.Precision.HIGHEST) + bo_ref[...]

    return pl.pallas_call(
        body,
        grid=(n // rb,),
        in_specs=[
            pl.BlockSpec((NC, rb, h), lambda r: (0, r, 0)),
            pl.BlockSpec((rb, NC), lambda r: (r, 0)),
            pl.BlockSpec((rb, h), lambda r: (r, 0)),
            pl.BlockSpec((1, h), lambda r: (0, 0)),
            pl.BlockSpec((h, c), lambda r: (0, 0)),
            pl.BlockSpec((1, c), lambda r: (0, 0)),
        ],
        out_specs=pl.BlockSpec((rb, c), lambda r: (r, 0)),
        out_shape=jax.ShapeDtypeStruct((n, c), jnp.float32),
    )(accp, degt, t_prev, b_prev, w_out, b_out)


def kernel(x, edge_index, edge_weight, W_in, b_in, W0, b0, W1, b1, W_out, b_out):
    n, _ = x.shape
    e = edge_weight.shape[0]
    rb = 1000 if n % 1000 == 0 else n // 8

    step = NW * CHUNK * 2
    ep = ((e + step - 1) // step) * step
    pad = ep - e
    if pad:
        fill = (jnp.arange(pad, dtype=jnp.int32) * 97) % n
        src = jnp.concatenate([edge_index[0], fill])
        dst = jnp.concatenate([edge_index[1], fill])
        ew = jnp.concatenate([edge_weight, jnp.zeros((pad,), edge_weight.dtype)])
    else:
        src, dst, ew = edge_index[0], edge_index[1], edge_weight

    ew_rep = jnp.repeat(ew, LANES)
    pidx = jnp.stack(
        [src.reshape(-1, CHUNK), dst.reshape(-1, CHUNK)], axis=1)

    b_in2 = b_in.reshape(1, -1)
    b02 = b0.reshape(1, -1)
    b12 = b1.reshape(1, -1)
    b_out2 = b_out.reshape(1, -1)

    np_ = ((n + NS * LANES - 1) // (NS * LANES)) * (NS * LANES)

    degp = _sc_degree(dst, ew, np_)
    degt = degp[:, :n].T

    t1, g1 = _tc_stage_a(x, degt, W_in, b_in2, W0, rb)
    acc1 = _sc_gather_scale_scatter(g1, pidx, ew_rep, np_, g1.shape[1])
    t2, g2 = _tc_stage_b(acc1[:, :n], degt, t1, b02, W1, rb)
    acc2 = _sc_gather_scale_scatter(g2, pidx, ew_rep, np_, g2.shape[1])
    return _tc_stage_c(acc2[:, :n], degt, t2, b12, W_out, b_out2, rb)


# speedup vs baseline: 13.7408x; 1.1081x over previous
import functools

import jax
import jax.numpy as jnp
from jax import lax
from jax.experimental import pallas as pl
from jax.experimental.pallas import tpu as pltpu
from jax.experimental.pallas import tpu_sc as plsc

NC = 2
NS = 16
NW = NC * NS
CHUNK = 128
LANES = 16


def _sc_degree(dst, ew, np_):
    ep = dst.shape[0]
    cpw = ep // (NW * CHUNK)
    per_sub = np_ // NS
    mesh = plsc.VectorSubcoreMesh(
        core_axis_name="c", subcore_axis_name="s", num_cores=NC, num_subcores=NS
    )

    @functools.partial(
        pl.kernel,
        out_type=jax.ShapeDtypeStruct((NC * np_,), jnp.float32),
        mesh=mesh,
        scratch_types=[
            pltpu.VMEM((CHUNK,), jnp.int32),
            pltpu.VMEM((CHUNK,), jnp.int32),
            pltpu.VMEM((CHUNK,), jnp.float32),
            pltpu.VMEM((CHUNK,), jnp.float32),
            pltpu.VMEM((per_sub,), jnp.float32),
            pltpu.VMEM_SHARED((np_,), jnp.float32),
            pltpu.SemaphoreType.DMA,
        ],
    )
    def deg_kernel(dst_hbm, ew_hbm, out_hbm, didx0, didx1, ewb0, ewb1,
                   zbuf, deg_sh, isem):
        c = lax.axis_index("c")
        s = lax.axis_index("s")
        wid = c * NS + s
        didxb = (didx0, didx1)
        ewbb = (ewb0, ewb1)

        @pl.loop(0, per_sub // LANES)
        def _(j):
            zbuf[pl.ds(j * LANES, LANES)] = jnp.zeros((LANES,), jnp.float32)

        sl = pl.ds(s * per_sub, per_sub)
        pltpu.sync_copy(zbuf, deg_sh.at[sl])
        plsc.subcore_barrier()

        def ld_start(i, b):
            base = (wid * cpw + i) * CHUNK
            pltpu.async_copy(dst_hbm.at[pl.ds(base, CHUNK)], didxb[b], isem)
            pltpu.async_copy(ew_hbm.at[pl.ds(base, CHUNK)], ewbb[b], isem)

        def ld_wait(i, b):
            base = (wid * cpw + i) * CHUNK
            pltpu.make_async_copy(dst_hbm.at[pl.ds(base, CHUNK)], didxb[b],
                                  isem).wait()
            pltpu.make_async_copy(ew_hbm.at[pl.ds(base, CHUNK)], ewbb[b],
                                  isem).wait()

        base0 = wid * cpw * CHUNK
        pltpu.sync_copy(dst_hbm.at[pl.ds(base0, CHUNK)], didx0)
        pltpu.sync_copy(ew_hbm.at[pl.ds(base0, CHUNK)], ewb0)

        @pl.loop(0, cpw // 2)
        def _(p):
            for b in (0, 1):
                i = p * 2 + b

                @pl.when(i > 0)
                def _():
                    ld_wait(i, b)

                @pl.when(i < cpw - 1)
                def _():
                    ld_start(i + 1, 1 - b)

                pltpu.sync_copy(ewbb[b], deg_sh.at[didxb[b]], add=True)

        plsc.subcore_barrier()
        pltpu.sync_copy(deg_sh.at[sl], zbuf)
        pltpu.sync_copy(zbuf, out_hbm.at[pl.ds(c * np_ + s * per_sub, per_sub)])

    return deg_kernel(dst, ew).reshape(NC, np_)


def _sc_gather_scale_scatter(g_tab, pidx, ew_rep, np_, d):
    cpw = pidx.shape[0] // NW
    per_sub = np_ // NS
    mesh = plsc.VectorSubcoreMesh(
        core_axis_name="c", subcore_axis_name="s", num_cores=NC, num_subcores=NS
    )

    @functools.partial(
        pl.kernel,
        out_type=jax.ShapeDtypeStruct((NC, np_, d), jnp.float32),
        mesh=mesh,
        scratch_types=[
            pltpu.VMEM((2, CHUNK), jnp.int32),
            pltpu.VMEM((2, CHUNK), jnp.int32),
            pltpu.VMEM((CHUNK * LANES,), jnp.float32),
            pltpu.VMEM((CHUNK * LANES,), jnp.float32),
            pltpu.VMEM((CHUNK, d), jnp.float32),
            pltpu.VMEM((CHUNK, d), jnp.float32),
            pltpu.VMEM_SHARED((np_, d), jnp.float32),
            pltpu.SemaphoreType.DMA,
            pltpu.SemaphoreType.DMA,
            pltpu.SemaphoreType.DMA,
        ],
    )
    def layer_kernel(g_hbm, pidx_hbm, ew_hbm, out_hbm,
                     idx0, idx1, wr0, wr1, rows0, rows1, acc_sh,
                     gsem0, gsem1, isem):
        c = lax.axis_index("c")
        s = lax.axis_index("s")
        wid = c * NS + s
        idxb = (idx0, idx1)
        wrb = (wr0, wr1)
        rowsb = (rows0, rows1)
        gsemb = (gsem0, gsem1)

        @pl.loop(0, CHUNK)
        def _(j):
            for k in range(d // LANES):
                rows0[j, pl.ds(k * LANES, LANES)] = jnp.zeros((LANES,), jnp.float32)

        for k in range(per_sub // CHUNK):
            pltpu.sync_copy(rows0, acc_sh.at[pl.ds(s * per_sub + k * CHUNK, CHUNK)])
        plsc.subcore_barrier()

        def idx_start(i, b):
            ch = wid * cpw + i
            pltpu.async_copy(pidx_hbm.at[ch], idxb[b], isem)
            pltpu.async_copy(
                ew_hbm.at[pl.ds(ch * CHUNK * LANES, CHUNK * LANES)], wrb[b], isem)

        def idx_wait(i, b):
            ch = wid * cpw + i
            pltpu.make_async_copy(pidx_hbm.at[ch], idxb[b], isem).wait()
            pltpu.make_async_copy(
                ew_hbm.at[pl.ds(ch * CHUNK * LANES, CHUNK * LANES)], wrb[b],
                isem).wait()

        def gather_start(b):
            pltpu.async_copy(g_hbm.at[idxb[b].at[0]], rowsb[b], gsemb[b])

        def gather_wait(b):
            pltpu.make_async_copy(g_hbm.at[idxb[b].at[0]], rowsb[b],
                                  gsemb[b]).wait()

        ch0 = wid * cpw
        pltpu.sync_copy(pidx_hbm.at[ch0], idx0)
        pltpu.sync_copy(ew_hbm.at[pl.ds(ch0 * CHUNK * LANES, CHUNK * LANES)], wr0)
        gather_start(0)
        idx_start(1, 1)

        @pl.loop(0, cpw // 2)
        def _(p):
            for b in (0, 1):
                i = p * 2 + b
                nb = 1 - b
                gather_wait(b)

                @pl.when(i < cpw - 1)
                def _():
                    idx_wait(i + 1, nb)
                    gather_start(nb)

                rows = rowsb[b]
                wr = wrb[b]

                @plsc.parallel_loop(0, CHUNK, unroll=4)
                def _(j):
                    wv = wr[pl.ds(j * LANES, LANES)]
                    for k in range(d // LANES):
                        slk = pl.ds(k * LANES, LANES)
                        rows[j, slk] = rows[j, slk] * wv

                pltpu.sync_copy(rows, acc_sh.at[idxb[b].at[1]], add=True)

                @pl.when(i < cpw - 2)
                def _():
                    idx_start(i + 2, b)

        plsc.subcore_barrier()
        out_core = out_hbm.at[c]
        for k in range(per_sub // CHUNK):
            sl = pl.ds(s * per_sub + k * CHUNK, CHUNK)
            pltpu.sync_copy(acc_sh.at[sl], out_core.at[sl])

    return layer_kernel(g_tab, pidx, ew_rep)


def _dinv_from_partials(deg2):
    deg = 1.0 + deg2[:, 0] + deg2[:, 1]
    return jnp.where(deg > 0, lax.rsqrt(deg), 0.0)


def _tc_stage_a1(x, w_in, b_in, w0, rb):
    n, din = x.shape
    h = w0.shape[1]

    def body(x_ref, wi_ref, bi_ref, w0_ref, t_ref):
        h1 = jnp.maximum(
            jnp.dot(x_ref[...], wi_ref[...], preferred_element_type=jnp.float32,
                    precision=lax.Precision.HIGHEST) + bi_ref[...], 0.0)
        t_ref[...] = jnp.dot(h1, w0_ref[...], preferred_element_type=jnp.float32,
                             precision=lax.Precision.HIGHEST)

    return pl.pallas_call(
        body,
        grid=(n // rb,),
        in_specs=[
            pl.BlockSpec((rb, din), lambda r: (r, 0)),
            pl.BlockSpec((din, h), lambda r: (0, 0)),
            pl.BlockSpec((1, h), lambda r: (0, 0)),
            pl.BlockSpec((h, h), lambda r: (0, 0)),
        ],
        out_specs=pl.BlockSpec((rb, h), lambda r: (r, 0)),
        out_shape=jax.ShapeDtypeStruct((n, h), jnp.float32),
    )(x, w_in, b_in, w0)


def _tc_stage_a2(degt, t1, rb):
    n, h = t1.shape

    def body(deg_ref, t_ref, g_ref):
        dinv = _dinv_from_partials(deg_ref[...])
        g_ref[...] = dinv[:, None] * t_ref[...]

    return pl.pallas_call(
        body,
        grid=(n // rb,),
        in_specs=[
            pl.BlockSpec((rb, NC), lambda r: (r, 0)),
            pl.BlockSpec((rb, h), lambda r: (r, 0)),
        ],
        out_specs=pl.BlockSpec((rb, h), lambda r: (r, 0)),
        out_shape=jax.ShapeDtypeStruct((n, h), jnp.float32),
    )(degt, t1)


def _tc_stage_b(accp, degt, t_prev, b_prev, w_next, rb):
    n, h = t_prev.shape
    hn = w_next.shape[1]

    def body(acc_ref, deg_ref, t_ref, b_ref, w_ref, tn_ref, gn_ref):
        dinv = _dinv_from_partials(deg_ref[...])
        acc = acc_ref[0] + acc_ref[1]
        hh = jnp.maximum(
            dinv[:, None] * acc + (dinv * dinv)[:, None] * t_ref[...] + b_ref[...],
            0.0)
        t = jnp.dot(hh, w_ref[...], preferred_element_type=jnp.float32,
                    precision=lax.Precision.HIGHEST)
        tn_ref[...] = t
        gn_ref[...] = dinv[:, None] * t

    return pl.pallas_call(
        body,
        grid=(n // rb,),
        in_specs=[
            pl.BlockSpec((NC, rb, h), lambda r: (0, r, 0)),
            pl.BlockSpec((rb, NC), lambda r: (r, 0)),
            pl.BlockSpec((rb, h), lambda r: (r, 0)),
            pl.BlockSpec((1, h), lambda r: (0, 0)),
            pl.BlockSpec((h, hn), lambda r: (0, 0)),
        ],
        out_specs=[
            pl.BlockSpec((rb, hn), lambda r: (r, 0)),
            pl.BlockSpec((rb, hn), lambda r: (r, 0)),
        ],
        out_shape=[
            jax.ShapeDtypeStruct((n, hn), jnp.float32),
            jax.ShapeDtypeStruct((n, hn), jnp.float32),
        ],
    )(accp, degt, t_prev, b_prev, w_next)


def _tc_stage_c(accp, degt, t_prev, b_prev, w_out, b_out, rb):
    n, h = t_prev.shape
    c = w_out.shape[1]

    def body(acc_ref, deg_ref, t_ref, b_ref, w_ref, bo_ref, o_ref):
        dinv = _dinv_from_partials(deg_ref[...])
        acc = acc_ref[0] + acc_ref[1]
        hh = jnp.maximum(
            dinv[:, None] * acc + (dinv * dinv)[:, None] * t_ref[...] + b_ref[...],
            0.0)
        o_ref[...] = jnp.dot(hh, w_ref[...], preferred_element_type=jnp.float32,
                             precision=lax.Precision.HIGHEST) + bo_ref[...]

    return pl.pallas_call(
        body,
        grid=(n // rb,),
        in_specs=[
            pl.BlockSpec((NC, rb, h), lambda r: (0, r, 0)),
            pl.BlockSpec((rb, NC), lambda r: (r, 0)),
            pl.BlockSpec((rb, h), lambda r: (r, 0)),
            pl.BlockSpec((1, h), lambda r: (0, 0)),
            pl.BlockSpec((h, c), lambda r: (0, 0)),
            pl.BlockSpec((1, c), lambda r: (0, 0)),
        ],
        out_specs=pl.BlockSpec((rb, c), lambda r: (r, 0)),
        out_shape=jax.ShapeDtypeStruct((n, c), jnp.float32),
    )(accp, degt, t_prev, b_prev, w_out, b_out)


def kernel(x, edge_index, edge_weight, W_in, b_in, W0, b0, W1, b1, W_out, b_out):
    n, _ = x.shape
    e = edge_weight.shape[0]
    rb = 1000 if n % 1000 == 0 else n // 8

    step = NW * CHUNK * 2
    ep = ((e + step - 1) // step) * step
    pad = ep - e
    if pad:
        fill = (jnp.arange(pad, dtype=jnp.int32) * 97) % n
        src = jnp.concatenate([edge_index[0], fill])
        dst = jnp.concatenate([edge_index[1], fill])
        ew = jnp.concatenate([edge_weight, jnp.zeros((pad,), edge_weight.dtype)])
    else:
        src, dst, ew = edge_index[0], edge_index[1], edge_weight

    ew_rep = jnp.repeat(ew, LANES)
    pidx = jnp.stack(
        [src.reshape(-1, CHUNK), dst.reshape(-1, CHUNK)], axis=1)

    b_in2 = b_in.reshape(1, -1)
    b02 = b0.reshape(1, -1)
    b12 = b1.reshape(1, -1)
    b_out2 = b_out.reshape(1, -1)

    np_ = ((n + NS * LANES - 1) // (NS * LANES)) * (NS * LANES)

    t1 = _tc_stage_a1(x, W_in, b_in2, W0, rb)
    degp = _sc_degree(dst, ew, np_)
    degt = degp[:, :n].T

    g1 = _tc_stage_a2(degt, t1, rb)
    acc1 = _sc_gather_scale_scatter(g1, pidx, ew_rep, np_, g1.shape[1])
    t2, g2 = _tc_stage_b(acc1[:, :n], degt, t1, b02, W1, rb)
    acc2 = _sc_gather_scale_scatter(g2, pidx, ew_rep, np_, g2.shape[1])
    return _tc_stage_c(acc2[:, :n], degt, t2, b12, W_out, b_out2, rb)


# speedup vs baseline: 14.2480x; 1.0369x over previous
import functools

import jax
import jax.numpy as jnp
from jax import lax
from jax.experimental import pallas as pl
from jax.experimental.pallas import tpu as pltpu
from jax.experimental.pallas import tpu_sc as plsc

NC = 2
NS = 16
NW = NC * NS
CHUNK = 128
LANES = 16


def _sc_degree(dst, ew, np_):
    ep = dst.shape[0]
    cpw = ep // (NW * CHUNK)
    per_sub = np_ // NS
    mesh = plsc.VectorSubcoreMesh(
        core_axis_name="c", subcore_axis_name="s", num_cores=NC, num_subcores=NS
    )

    @functools.partial(
        pl.kernel,
        out_type=jax.ShapeDtypeStruct((NC * np_,), jnp.float32),
        mesh=mesh,
        scratch_types=[
            pltpu.VMEM((CHUNK,), jnp.int32),
            pltpu.VMEM((CHUNK,), jnp.int32),
            pltpu.VMEM((CHUNK,), jnp.float32),
            pltpu.VMEM((CHUNK,), jnp.float32),
            pltpu.VMEM((per_sub,), jnp.float32),
            pltpu.VMEM_SHARED((np_,), jnp.float32),
            pltpu.SemaphoreType.DMA,
        ],
    )
    def deg_kernel(dst_hbm, ew_hbm, out_hbm, didx0, didx1, ewb0, ewb1,
                   zbuf, deg_sh, isem):
        c = lax.axis_index("c")
        s = lax.axis_index("s")
        wid = c * NS + s
        didxb = (didx0, didx1)
        ewbb = (ewb0, ewb1)

        @pl.loop(0, per_sub // LANES)
        def _(j):
            zbuf[pl.ds(j * LANES, LANES)] = jnp.zeros((LANES,), jnp.float32)

        sl = pl.ds(s * per_sub, per_sub)
        pltpu.sync_copy(zbuf, deg_sh.at[sl])
        plsc.subcore_barrier()

        def ld_start(i, b):
            base = (wid * cpw + i) * CHUNK
            pltpu.async_copy(dst_hbm.at[pl.ds(base, CHUNK)], didxb[b], isem)
            pltpu.async_copy(ew_hbm.at[pl.ds(base, CHUNK)], ewbb[b], isem)

        def ld_wait(i, b):
            base = (wid * cpw + i) * CHUNK
            pltpu.make_async_copy(dst_hbm.at[pl.ds(base, CHUNK)], didxb[b],
                                  isem).wait()
            pltpu.make_async_copy(ew_hbm.at[pl.ds(base, CHUNK)], ewbb[b],
                                  isem).wait()

        base0 = wid * cpw * CHUNK
        pltpu.sync_copy(dst_hbm.at[pl.ds(base0, CHUNK)], didx0)
        pltpu.sync_copy(ew_hbm.at[pl.ds(base0, CHUNK)], ewb0)

        @pl.loop(0, cpw // 2)
        def _(p):
            for b in (0, 1):
                i = p * 2 + b

                @pl.when(i > 0)
                def _():
                    ld_wait(i, b)

                @pl.when(i < cpw - 1)
                def _():
                    ld_start(i + 1, 1 - b)

                pltpu.sync_copy(ewbb[b], deg_sh.at[didxb[b]], add=True)

        plsc.subcore_barrier()
        pltpu.sync_copy(deg_sh.at[sl], zbuf)
        pltpu.sync_copy(zbuf, out_hbm.at[pl.ds(c * np_ + s * per_sub, per_sub)])

    return deg_kernel(dst, ew).reshape(NC, np_)


def _sc_gather_scale_scatter(g_tab, pidx, ew_rep, np_, d):
    cpw = pidx.shape[0] // NW
    per_sub = np_ // NS
    mesh = plsc.VectorSubcoreMesh(
        core_axis_name="c", subcore_axis_name="s", num_cores=NC, num_subcores=NS
    )

    @functools.partial(
        pl.kernel,
        out_type=jax.ShapeDtypeStruct((NC, np_, d), jnp.float32),
        mesh=mesh,
        scratch_types=[
            pltpu.VMEM((2, CHUNK), jnp.int32),
            pltpu.VMEM((2, CHUNK), jnp.int32),
            pltpu.VMEM((CHUNK * LANES,), jnp.float32),
            pltpu.VMEM((CHUNK * LANES,), jnp.float32),
            pltpu.VMEM((CHUNK, d), jnp.float32),
            pltpu.VMEM((CHUNK, d), jnp.float32),
            pltpu.VMEM_SHARED((np_, d), jnp.float32),
            pltpu.SemaphoreType.DMA,
            pltpu.SemaphoreType.DMA,
            pltpu.SemaphoreType.DMA,
        ],
    )
    def layer_kernel(g_hbm, pidx_hbm, ew_hbm, out_hbm,
                     idx0, idx1, wr0, wr1, rows0, rows1, acc_sh,
                     gsem0, gsem1, isem):
        c = lax.axis_index("c")
        s = lax.axis_index("s")
        wid = c * NS + s
        idxb = (idx0, idx1)
        wrb = (wr0, wr1)
        rowsb = (rows0, rows1)
        gsemb = (gsem0, gsem1)

        @pl.loop(0, CHUNK)
        def _(j):
            for k in range(d // LANES):
                rows0[j, pl.ds(k * LANES, LANES)] = jnp.zeros((LANES,), jnp.float32)

        for k in range(per_sub // CHUNK):
            pltpu.sync_copy(rows0, acc_sh.at[pl.ds(s * per_sub + k * CHUNK, CHUNK)])
        plsc.subcore_barrier()

        def idx_start(i, b):
            ch = wid * cpw + i
            pltpu.async_copy(pidx_hbm.at[ch], idxb[b], isem)
            pltpu.async_copy(
                ew_hbm.at[pl.ds(ch * CHUNK * LANES, CHUNK * LANES)], wrb[b], isem)

        def idx_wait(i, b):
            ch = wid * cpw + i
            pltpu.make_async_copy(pidx_hbm.at[ch], idxb[b], isem).wait()
            pltpu.make_async_copy(
                ew_hbm.at[pl.ds(ch * CHUNK * LANES, CHUNK * LANES)], wrb[b],
                isem).wait()

        def gather_start(b):
            pltpu.async_copy(g_hbm.at[idxb[b].at[0]], rowsb[b], gsemb[b])

        def gather_wait(b):
            pltpu.make_async_copy(g_hbm.at[idxb[b].at[0]], rowsb[b],
                                  gsemb[b]).wait()

        ch0 = wid * cpw
        pltpu.sync_copy(pidx_hbm.at[ch0], idx0)
        pltpu.sync_copy(ew_hbm.at[pl.ds(ch0 * CHUNK * LANES, CHUNK * LANES)], wr0)
        gather_start(0)
        idx_start(1, 1)

        @pl.loop(0, cpw // 2)
        def _(p):
            for b in (0, 1):
                i = p * 2 + b
                nb = 1 - b
                gather_wait(b)

                @pl.when(i < cpw - 1)
                def _():
                    idx_wait(i + 1, nb)
                    gather_start(nb)

                rows = rowsb[b]
                wr = wrb[b]

                @plsc.parallel_loop(0, CHUNK, unroll=4)
                def _(j):
                    wv = wr[pl.ds(j * LANES, LANES)]
                    for k in range(d // LANES):
                        slk = pl.ds(k * LANES, LANES)
                        rows[j, slk] = rows[j, slk] * wv

                pltpu.sync_copy(rows, acc_sh.at[idxb[b].at[1]], add=True)

                @pl.when(i < cpw - 2)
                def _():
                    idx_start(i + 2, b)

        plsc.subcore_barrier()
        out_core = out_hbm.at[c]
        for k in range(per_sub // CHUNK):
            sl = pl.ds(s * per_sub + k * CHUNK, CHUNK)
            pltpu.sync_copy(acc_sh.at[sl], out_core.at[sl])

    return layer_kernel(g_tab, pidx, ew_rep)


def _dinv_from_partials(deg2):
    deg = 1.0 + deg2[:, 0] + deg2[:, 1]
    return jnp.where(deg > 0, lax.rsqrt(deg), 0.0)


def _tc_stage_a1(x, w_in, b_in, w0, rb):
    n, din = x.shape
    h = w0.shape[1]

    def body(x_ref, wi_ref, bi_ref, w0_ref, t_ref):
        h1 = jnp.maximum(
            jnp.dot(x_ref[...], wi_ref[...], preferred_element_type=jnp.float32,
                    precision=lax.Precision.HIGHEST) + bi_ref[...], 0.0)
        t_ref[...] = jnp.dot(h1, w0_ref[...], preferred_element_type=jnp.float32,
                             precision=lax.Precision.HIGHEST)

    return pl.pallas_call(
        body,
        grid=(n // rb,),
        in_specs=[
            pl.BlockSpec((rb, din), lambda r: (r, 0)),
            pl.BlockSpec((din, h), lambda r: (0, 0)),
            pl.BlockSpec((1, h), lambda r: (0, 0)),
            pl.BlockSpec((h, h), lambda r: (0, 0)),
        ],
        out_specs=pl.BlockSpec((rb, h), lambda r: (r, 0)),
        out_shape=jax.ShapeDtypeStruct((n, h), jnp.float32),
    )(x, w_in, b_in, w0)


def _tc_stage_a2(degt, t1, rb):
    n, h = t1.shape

    def body(deg_ref, t_ref, g_ref):
        dinv = _dinv_from_partials(deg_ref[...])
        g_ref[...] = dinv[:, None] * t_ref[...]

    return pl.pallas_call(
        body,
        grid=(n // rb,),
        in_specs=[
            pl.BlockSpec((rb, NC), lambda r: (r, 0)),
            pl.BlockSpec((rb, h), lambda r: (r, 0)),
        ],
        out_specs=pl.BlockSpec((rb, h), lambda r: (r, 0)),
        out_shape=jax.ShapeDtypeStruct((n, h), jnp.float32),
    )(degt, t1)


def _tc_stage_b(accp, degt, t_prev, b_prev, w_next, rb):
    n, h = t_prev.shape
    hn = w_next.shape[1]

    def body(acc_ref, deg_ref, t_ref, b_ref, w_ref, tn_ref, gn_ref):
        dinv = _dinv_from_partials(deg_ref[...])
        acc = acc_ref[0] + acc_ref[1]
        hh = jnp.maximum(
            dinv[:, None] * acc + (dinv * dinv)[:, None] * t_ref[...] + b_ref[...],
            0.0)
        t = jnp.dot(hh, w_ref[...], preferred_element_type=jnp.float32,
                    precision=lax.Precision.HIGHEST)
        tn_ref[...] = t
        gn_ref[...] = dinv[:, None] * t

    return pl.pallas_call(
        body,
        grid=(n // rb,),
        in_specs=[
            pl.BlockSpec((NC, rb, h), lambda r: (0, r, 0)),
            pl.BlockSpec((rb, NC), lambda r: (r, 0)),
            pl.BlockSpec((rb, h), lambda r: (r, 0)),
            pl.BlockSpec((1, h), lambda r: (0, 0)),
            pl.BlockSpec((h, hn), lambda r: (0, 0)),
        ],
        out_specs=[
            pl.BlockSpec((rb, hn), lambda r: (r, 0)),
            pl.BlockSpec((rb, hn), lambda r: (r, 0)),
        ],
        out_shape=[
            jax.ShapeDtypeStruct((n, hn), jnp.float32),
            jax.ShapeDtypeStruct((n, hn), jnp.float32),
        ],
    )(accp, degt, t_prev, b_prev, w_next)


def _tc_stage_c(accp, degt, t_prev, b_prev, w_out, b_out, rb):
    n, h = t_prev.shape
    c = w_out.shape[1]

    def body(acc_ref, deg_ref, t_ref, b_ref, w_ref, bo_ref, o_ref):
        dinv = _dinv_from_partials(deg_ref[...])
        acc = acc_ref[0] + acc_ref[1]
        hh = jnp.maximum(
            dinv[:, None] * acc + (dinv * dinv)[:, None] * t_ref[...] + b_ref[...],
            0.0)
        o_ref[...] = jnp.dot(hh, w_ref[...], preferred_element_type=jnp.float32,
                             precision=lax.Precision.HIGHEST) + bo_ref[...]

    return pl.pallas_call(
        body,
        grid=(n // rb,),
        in_specs=[
            pl.BlockSpec((NC, rb, h), lambda r: (0, r, 0)),
            pl.BlockSpec((rb, NC), lambda r: (r, 0)),
            pl.BlockSpec((rb, h), lambda r: (r, 0)),
            pl.BlockSpec((1, h), lambda r: (0, 0)),
            pl.BlockSpec((h, c), lambda r: (0, 0)),
            pl.BlockSpec((1, c), lambda r: (0, 0)),
        ],
        out_specs=pl.BlockSpec((rb, c), lambda r: (r, 0)),
        out_shape=jax.ShapeDtypeStruct((n, c), jnp.float32),
    )(accp, degt, t_prev, b_prev, w_out, b_out)


def kernel(x, edge_index, edge_weight, W_in, b_in, W0, b0, W1, b1, W_out, b_out):
    n, _ = x.shape
    e = edge_weight.shape[0]
    rb = next((b for b in (2000, 1000, 504, 8) if n % b == 0), n)

    step = NW * CHUNK * 2
    ep = ((e + step - 1) // step) * step
    pad = ep - e
    if pad:
        fill = (jnp.arange(pad, dtype=jnp.int32) * 97) % n
        src = jnp.concatenate([edge_index[0], fill])
        dst = jnp.concatenate([edge_index[1], fill])
        ew = jnp.concatenate([edge_weight, jnp.zeros((pad,), edge_weight.dtype)])
    else:
        src, dst, ew = edge_index[0], edge_index[1], edge_weight

    ew_rep = jnp.repeat(ew, LANES)
    pidx = jnp.stack(
        [src.reshape(-1, CHUNK), dst.reshape(-1, CHUNK)], axis=1)

    b_in2 = b_in.reshape(1, -1)
    b02 = b0.reshape(1, -1)
    b12 = b1.reshape(1, -1)
    b_out2 = b_out.reshape(1, -1)

    np_ = ((n + NS * LANES - 1) // (NS * LANES)) * (NS * LANES)

    t1 = _tc_stage_a1(x, W_in, b_in2, W0, rb)
    degp = _sc_degree(dst, ew, np_)
    degt = degp[:, :n].T

    g1 = _tc_stage_a2(degt, t1, rb)
    acc1 = _sc_gather_scale_scatter(g1, pidx, ew_rep, np_, g1.shape[1])
    t2, g2 = _tc_stage_b(acc1[:, :n], degt, t1, b02, W1, rb)
    acc2 = _sc_gather_scale_scatter(g2, pidx, ew_rep, np_, g2.shape[1])
    return _tc_stage_c(acc2[:, :n], degt, t2, b12, W_out, b_out2, rb)
